# baseline (speedup 1.0000x reference)
import functools

import jax
import jax.numpy as jnp
from jax.experimental import pallas as pl
from jax.experimental.pallas import tpu as pltpu

_TOP_K = 8
_ROW_TILE = 1024
_CHUNK = 128


def _gate_rows(logits):
    neg_inf = jnp.float32(-jnp.inf)
    k_f = jnp.float32(_TOP_K)
    n_exp = logits.shape[-1]
    cur = logits
    cnt = jnp.zeros(logits.shape[:-1] + (1,), jnp.float32)
    t = jnp.full(logits.shape[:-1] + (1,), neg_inf)
    row_max = None
    for it in range(_TOP_K):
        m = jnp.max(cur, axis=-1, keepdims=True)
        if it == 0:
            row_max = m
        eq = cur == m
        t = jnp.where(cnt < k_f, m, t)
        cnt = cnt + jnp.sum(jnp.where(eq, 1.0, 0.0), axis=-1, keepdims=True)
        cur = jnp.where(eq, neg_inf, cur)
    gt = logits > t
    eqt = jnp.where(logits == t, 1.0, 0.0)
    n_gt = jnp.sum(jnp.where(gt, 1.0, 0.0), axis=-1, keepdims=True)
    ri = jax.lax.broadcasted_iota(jnp.int32, (n_exp, n_exp), 0)
    ci = jax.lax.broadcasted_iota(jnp.int32, (n_exp, n_exp), 1)
    tri = jnp.where(ri <= ci, 1.0, 0.0)
    rank_eq = jnp.dot(eqt, tri, preferred_element_type=jnp.float32)
    keep = jnp.logical_or(gt, (eqt > 0.0) & (rank_eq <= k_f - n_gt))
    masked = jnp.where(keep, logits, 0.0)
    mx = jnp.maximum(row_max, 0.0)
    e = jnp.exp(masked - mx)
    return e / jnp.sum(e, axis=-1, keepdims=True)


def _gating_body(x_ref, w_ref, b_ref, o_ref):
    for c in range(_ROW_TILE // _CHUNK):
        rows = pl.ds(c * _CHUNK, _CHUNK)
        logits = jnp.dot(x_ref[rows, :], w_ref[...],
                         preferred_element_type=jnp.float32) + b_ref[...]
        o_ref[rows, :] = _gate_rows(logits)


@jax.jit
def kernel(x, W, b):
    n_tok, d_model = x.shape
    n_exp = W.shape[1]
    b2 = b.reshape(1, n_exp)
    grid = (n_tok // _ROW_TILE,)
    return pl.pallas_call(
        _gating_body,
        grid=grid,
        in_specs=[
            pl.BlockSpec((_ROW_TILE, d_model), lambda i: (i, 0)),
            pl.BlockSpec((d_model, n_exp), lambda i: (0, 0)),
            pl.BlockSpec((1, n_exp), lambda i: (0, 0)),
        ],
        out_specs=pl.BlockSpec((_ROW_TILE, n_exp), lambda i: (i, 0)),
        out_shape=jax.ShapeDtypeStruct((n_tok, n_exp), jnp.float32),
        compiler_params=pltpu.CompilerParams(
            dimension_semantics=("arbitrary",),
        ),
    )(x, W, b2)


# speedup vs baseline: 1.6139x; 1.6139x over previous
import functools

import jax
import jax.numpy as jnp
from jax.experimental import pallas as pl
from jax.experimental.pallas import tpu as pltpu

_TOP_K = 8
_ROW_TILE = 1024
_CHUNK = 256


def _gate_rows(logits):
    neg_inf = jnp.float32(-jnp.inf)
    k_f = jnp.float32(_TOP_K)
    n_exp = logits.shape[-1]
    cur = logits
    cnt = jnp.zeros(logits.shape[:-1] + (1,), jnp.float32)
    t = jnp.full(logits.shape[:-1] + (1,), neg_inf)
    row_max = None
    for it in range(_TOP_K):
        m = jnp.max(cur, axis=-1, keepdims=True)
        if it == 0:
            row_max = m
        eq = cur == m
        t = jnp.where(cnt < k_f, m, t)
        cnt = cnt + jnp.sum(jnp.where(eq, 1.0, 0.0), axis=-1, keepdims=True)
        cur = jnp.where(eq, neg_inf, cur)
    gt = logits > t
    eqt = jnp.where(logits == t, 1.0, 0.0)
    n_gt = jnp.sum(jnp.where(gt, 1.0, 0.0), axis=-1, keepdims=True)
    ri = jax.lax.broadcasted_iota(jnp.int32, (n_exp, n_exp), 0)
    ci = jax.lax.broadcasted_iota(jnp.int32, (n_exp, n_exp), 1)
    tri = jnp.where(ri <= ci, 1.0, 0.0)
    rank_eq = jnp.dot(eqt, tri, preferred_element_type=jnp.float32)
    keep = jnp.logical_or(gt, (eqt > 0.0) & (rank_eq <= k_f - n_gt))
    masked = jnp.where(keep, logits, 0.0)
    mx = jnp.maximum(row_max, 0.0)
    e = jnp.exp(masked - mx)
    return e / jnp.sum(e, axis=-1, keepdims=True)


def _matmul_chunk(x_ref, w_ref, b_ref, c):
    rows = pl.ds(c * _CHUNK, _CHUNK)
    return jnp.dot(x_ref[rows, :], w_ref[...],
                   preferred_element_type=jnp.float32) + b_ref[...]


def _gating_body(x_ref, w_ref, b_ref, o_ref):
    n_chunks = _ROW_TILE // _CHUNK
    logits = _matmul_chunk(x_ref, w_ref, b_ref, 0)
    for c in range(n_chunks):
        cur_logits = logits
        if c + 1 < n_chunks:
            logits = _matmul_chunk(x_ref, w_ref, b_ref, c + 1)
        o_ref[pl.ds(c * _CHUNK, _CHUNK), :] = _gate_rows(cur_logits)


@jax.jit
def kernel(x, W, b):
    n_tok, d_model = x.shape
    n_exp = W.shape[1]
    b2 = b.reshape(1, n_exp)
    grid = (n_tok // _ROW_TILE,)
    return pl.pallas_call(
        _gating_body,
        grid=grid,
        in_specs=[
            pl.BlockSpec((_ROW_TILE, d_model), lambda i: (i, 0)),
            pl.BlockSpec((d_model, n_exp), lambda i: (0, 0)),
            pl.BlockSpec((1, n_exp), lambda i: (0, 0)),
        ],
        out_specs=pl.BlockSpec((_ROW_TILE, n_exp), lambda i: (i, 0)),
        out_shape=jax.ShapeDtypeStruct((n_tok, n_exp), jnp.float32),
        compiler_params=pltpu.CompilerParams(
            dimension_semantics=("arbitrary",),
        ),
    )(x, W, b2)
